# Initial kernel scaffold; baseline (speedup 1.0000x reference)
import functools

import jax
import jax.numpy as jnp
from jax import lax
from jax.experimental import pallas as pl
from jax.experimental.pallas import tpu as pltpu
from jax.experimental.pallas import tpu_sc as plsc

N = 10000
E = 320000
ET = E + N
D_IN = 128
HID = 128
D_OUT = 64
H = 8

NC, NS = 2, 16
NW = NC * NS
K = 32
CPW = 323
EPW = CPW * K
EP = NW * EPW
RPT = N // NS

_mesh = plsc.VectorSubcoreMesh(core_axis_name="c", subcore_axis_name="s")


_R = 400


def _proj_compute(C, xb, wl_ref, wr_ref, attf_ref, xl_ref, xr_ref, sb_ref):
    xl = jnp.dot(xb, wl_ref[...], preferred_element_type=jnp.float32)
    xr = jnp.dot(xb, wr_ref[...], preferred_element_type=jnp.float32)
    s = xl + xr
    lr = jnp.where(s > 0.0, s, 0.2 * s) * attf_ref[...]
    cols = [jnp.sum(lr[:, h * C:(h + 1) * C], axis=1, keepdims=True)
            for h in range(H)]
    pad = jnp.zeros((xb.shape[0], 16 - H), jnp.float32)
    xl_ref[...] = xl
    xr_ref[...] = xr
    sb_ref[...] = jnp.concatenate(cols + [pad], axis=1)


def _proj1_body(C, x_ref, wl_ref, wr_ref, attf_ref, xl_ref, xr_ref, sb_ref):
    _proj_compute(C, x_ref[...], wl_ref, wr_ref, attf_ref,
                  xl_ref, xr_ref, sb_ref)


def _proj1(x, wl, wr, attf):
    D = wl.shape[1]
    C = D // H
    return pl.pallas_call(
        functools.partial(_proj1_body, C),
        grid=(N // _R,),
        in_specs=[
            pl.BlockSpec((_R, x.shape[1]), lambda i: (i, 0)),
            pl.BlockSpec(wl.shape, lambda i: (0, 0)),
            pl.BlockSpec(wr.shape, lambda i: (0, 0)),
            pl.BlockSpec((1, D), lambda i: (0, 0)),
        ],
        out_specs=[
            pl.BlockSpec((_R, D), lambda i: (i, 0)),
            pl.BlockSpec((_R, D), lambda i: (i, 0)),
            pl.BlockSpec((_R, 16), lambda i: (i, 0)),
        ],
        out_shape=[
            jax.ShapeDtypeStruct((N, D), jnp.float32),
            jax.ShapeDtypeStruct((N, D), jnp.float32),
            jax.ShapeDtypeStruct((N, 16), jnp.float32),
        ],
    )(x, wl, wr, attf)


def _proj2_body(C, ms0_ref, ms1_ref, b_ref, wl_ref, wr_ref, attf_ref,
                xl_ref, xr_ref, sb_ref):
    h1 = (ms0_ref[...] + ms1_ref[...]) * (1.0 / H) + b_ref[...]
    h1 = jnp.maximum(h1, 0.0)
    _proj_compute(C, h1, wl_ref, wr_ref, attf_ref, xl_ref, xr_ref, sb_ref)


def _proj2(ms0, ms1, b, wl, wr, attf):
    Cin = ms0.shape[1]
    D = wl.shape[1]
    C = D // H
    return pl.pallas_call(
        functools.partial(_proj2_body, C),
        grid=(N // _R,),
        in_specs=[
            pl.BlockSpec((_R, Cin), lambda i: (i, 0)),
            pl.BlockSpec((_R, Cin), lambda i: (i, 0)),
            pl.BlockSpec((1, Cin), lambda i: (0, 0)),
            pl.BlockSpec(wl.shape, lambda i: (0, 0)),
            pl.BlockSpec(wr.shape, lambda i: (0, 0)),
            pl.BlockSpec((1, D), lambda i: (0, 0)),
        ],
        out_specs=[
            pl.BlockSpec((_R, D), lambda i: (i, 0)),
            pl.BlockSpec((_R, D), lambda i: (i, 0)),
            pl.BlockSpec((_R, 16), lambda i: (i, 0)),
        ],
        out_shape=[
            jax.ShapeDtypeStruct((N, D), jnp.float32),
            jax.ShapeDtypeStruct((N, D), jnp.float32),
            jax.ShapeDtypeStruct((N, 16), jnp.float32),
        ],
    )(ms0, ms1, b.reshape(1, -1), wl, wr, attf)


def _final_body(ms0_ref, ms1_ref, b_ref, out_ref):
    h2 = (ms0_ref[...] + ms1_ref[...]) * (1.0 / H) + b_ref[...]
    m = jnp.max(h2, axis=1, keepdims=True)
    z = h2 - m
    lse = jnp.log(jnp.sum(jnp.exp(z), axis=1, keepdims=True))
    out_ref[...] = z - lse


def _final(ms0, ms1, b):
    C = ms0.shape[1]
    return pl.pallas_call(
        _final_body,
        grid=(N // _R,),
        in_specs=[
            pl.BlockSpec((_R, C), lambda i: (i, 0)),
            pl.BlockSpec((_R, C), lambda i: (i, 0)),
            pl.BlockSpec((1, C), lambda i: (0, 0)),
        ],
        out_specs=pl.BlockSpec((_R, C), lambda i: (i, 0)),
        out_shape=jax.ShapeDtypeStruct((N, C), jnp.float32),
    )(ms0, ms1, b.reshape(1, -1))


def _make_edge_pass1(D):
    C = D // H
    CB = C // 16

    @functools.partial(
        pl.kernel,
        out_type=(
            jax.ShapeDtypeStruct((EP, 16), jnp.float32),
            jax.ShapeDtypeStruct((N, 16), jnp.float32),
            jax.ShapeDtypeStruct((N, 16), jnp.float32),
        ),
        mesh=_mesh,
        scratch_types=[
            pltpu.VMEM((K,), jnp.int32),
            pltpu.VMEM((K,), jnp.int32),
            pltpu.VMEM((K, D), jnp.float32),
            pltpu.VMEM((K, D), jnp.float32),
            pltpu.VMEM((K, 16), jnp.float32),
            pltpu.VMEM((K, 16), jnp.float32),
            pltpu.VMEM((D,), jnp.float32),
            pltpu.VMEM((125, 16), jnp.float32),
            pltpu.VMEM_SHARED((N, 16), jnp.float32),
            pltpu.SemaphoreType.DMA,
        ],
    )
    def kfn(xl_hbm, xr_hbm, sb_hbm, att_hbm, src_hbm, dst_hbm,
            ex_hbm, den0_hbm, den1_hbm,
            srcv, dstv, xlr, xrr, sbr, exb, attv, zbuf, den_acc, sem):
        cid = lax.axis_index("c")
        sid = lax.axis_index("s")
        wid = sid * NC + cid
        r0 = sid * RPT

        def zrow(i, _):
            zbuf[i, :] = jnp.zeros((16,), jnp.float32)
            return 0
        lax.fori_loop(0, 125, zrow, 0)
        for j in range(RPT // 125):
            pltpu.sync_copy(zbuf, den_acc.at[pl.ds(r0 + j * 125, 125)])
        pltpu.sync_copy(att_hbm, attv)
        plsc.subcore_barrier()

        iot = lax.broadcasted_iota(jnp.int32, (16,), 0)
        lanemask = iot < H

        def chunk(g, _):
            base = wid * EPW + g * K
            pltpu.sync_copy(src_hbm.at[pl.ds(base, K)], srcv)
            pltpu.sync_copy(dst_hbm.at[pl.ds(base, K)], dstv)
            pltpu.async_copy(xl_hbm.at[srcv], xlr, sem).wait()
            pltpu.async_copy(xr_hbm.at[dstv], xrr, sem).wait()
            pltpu.async_copy(sb_hbm.at[dstv], sbr, sem).wait()

            def edge(i, _):
                av = jnp.zeros((16,), jnp.float32)
                for h in range(H):
                    acc = jnp.zeros((16,), jnp.float32)
                    for j in range(CB):
                        off = h * C + j * 16
                        s = xlr[i, pl.ds(off, 16)] + xrr[i, pl.ds(off, 16)]
                        s = jnp.where(s > 0.0, s, 0.2 * s)
                        acc = acc + s * attv[pl.ds(off, 16)]
                    ah = jnp.sum(acc)
                    av = jnp.where(iot == h, ah, av)
                exv = jnp.exp(av - sbr[i, :])
                valid = (base + i) < ET
                exv = jnp.where(jnp.logical_and(lanemask, valid), exv, 0.0)
                exb[i, :] = exv
                return 0
            lax.fori_loop(0, K, edge, 0)
            pltpu.sync_copy(exb, ex_hbm.at[pl.ds(base, K)])
            pltpu.sync_copy(exb, den_acc.at[dstv], add=True)
            return 0
        lax.fori_loop(0, CPW, chunk, 0)
        plsc.subcore_barrier()

        @pl.when(cid == 0)
        def _():
            pltpu.sync_copy(den_acc.at[pl.ds(r0, RPT)],
                            den0_hbm.at[pl.ds(r0, RPT)])

        @pl.when(cid == 1)
        def _():
            pltpu.sync_copy(den_acc.at[pl.ds(r0, RPT)],
                            den1_hbm.at[pl.ds(r0, RPT)])

    return kfn


def _make_edge_pass2(D):
    C = D // H
    CB = C // 16

    @functools.partial(
        pl.kernel,
        out_type=(
            jax.ShapeDtypeStruct((EP, 16), jnp.float32),
            jax.ShapeDtypeStruct((N, C), jnp.float32),
            jax.ShapeDtypeStruct((N, C), jnp.float32),
        ),
        mesh=_mesh,
        scratch_types=[
            pltpu.VMEM((K,), jnp.int32),
            pltpu.VMEM((K,), jnp.int32),
            pltpu.VMEM((K, D), jnp.float32),
            pltpu.VMEM((K, 16), jnp.float32),
            pltpu.VMEM((K, 16), jnp.float32),
            pltpu.VMEM((K, 16), jnp.float32),
            pltpu.VMEM((K, 16), jnp.float32),
            pltpu.VMEM((K, C), jnp.float32),
            pltpu.VMEM((125, C), jnp.float32),
            pltpu.VMEM_SHARED((N, C), jnp.float32),
            pltpu.SemaphoreType.DMA,
        ],
    )
    def kfn(xl_hbm, ex_hbm, den0_hbm, den1_hbm, src_hbm, dst_hbm,
            a_hbm, ms0_hbm, ms1_hbm,
            srcv, dstv, xlr, exb, d0r, d1r, ab, mb, zbuf, m_acc, sem):
        cid = lax.axis_index("c")
        sid = lax.axis_index("s")
        wid = sid * NC + cid
        r0 = sid * RPT

        def zrow(i, _):
            for j in range(CB):
                zbuf[i, pl.ds(j * 16, 16)] = jnp.zeros((16,), jnp.float32)
            return 0
        lax.fori_loop(0, 125, zrow, 0)
        for j in range(RPT // 125):
            pltpu.sync_copy(zbuf, m_acc.at[pl.ds(r0 + j * 125, 125)])
        plsc.subcore_barrier()

        def chunk(g, _):
            base = wid * EPW + g * K
            pltpu.sync_copy(src_hbm.at[pl.ds(base, K)], srcv)
            pltpu.sync_copy(dst_hbm.at[pl.ds(base, K)], dstv)
            pltpu.async_copy(xl_hbm.at[srcv], xlr, sem).wait()
            pltpu.sync_copy(ex_hbm.at[pl.ds(base, K)], exb)
            pltpu.async_copy(den0_hbm.at[dstv], d0r, sem).wait()
            pltpu.async_copy(den1_hbm.at[dstv], d1r, sem).wait()

            def edge(i, _):
                den = d0r[i, :] + d1r[i, :]
                a = exb[i, :] / (den + 1e-16)
                ab[i, :] = a
                for j in range(CB):
                    acc = jnp.zeros((16,), jnp.float32)
                    for h in range(H):
                        off = h * C + j * 16
                        acc = acc + ab[i, h] * xlr[i, pl.ds(off, 16)]
                    mb[i, pl.ds(j * 16, 16)] = acc
                return 0
            lax.fori_loop(0, K, edge, 0)
            pltpu.sync_copy(ab, a_hbm.at[pl.ds(base, K)])
            pltpu.sync_copy(mb, m_acc.at[dstv], add=True)
            return 0
        lax.fori_loop(0, CPW, chunk, 0)
        plsc.subcore_barrier()

        @pl.when(cid == 0)
        def _():
            pltpu.sync_copy(m_acc.at[pl.ds(r0, RPT)],
                            ms0_hbm.at[pl.ds(r0, RPT)])

        @pl.when(cid == 1)
        def _():
            pltpu.sync_copy(m_acc.at[pl.ds(r0, RPT)],
                            ms1_hbm.at[pl.ds(r0, RPT)])

    return kfn


_edge1_l1 = _make_edge_pass1(H * HID)
_edge2_l1 = _make_edge_pass2(H * HID)
_edge1_l2 = _make_edge_pass1(H * D_OUT)
_edge2_l2 = _make_edge_pass2(H * D_OUT)


def kernel(x, edge_index, Wl1, Wr1, att1, b1, Wl2, Wr2, att2, b2):
    loops = jnp.arange(N, dtype=edge_index.dtype)
    ei = jnp.concatenate([edge_index, jnp.stack([loops, loops])], axis=1)
    src = jnp.pad(ei[0], (0, EP - ET)).astype(jnp.int32)
    dst = jnp.pad(ei[1], (0, EP - ET)).astype(jnp.int32)

    attf1 = att1.reshape(1, H * HID)
    xl1, xr1, sb1 = _proj1(x, Wl1, Wr1, attf1)
    ex1, den10, den11 = _edge1_l1(xl1, xr1, sb1, att1.reshape(-1), src, dst)
    a1f, ms10, ms11 = _edge2_l1(xl1, ex1, den10, den11, src, dst)

    attf2 = att2.reshape(1, H * D_OUT)
    xl2, xr2, sb2 = _proj2(ms10, ms11, b1, Wl2, Wr2, attf2)
    ex2, den20, den21 = _edge1_l2(xl2, xr2, sb2, att2.reshape(-1), src, dst)
    a2f, ms20, ms21 = _edge2_l2(xl2, ex2, den20, den21, src, dst)

    out = _final(ms20, ms21, b2)
    return out, ei, a1f[:ET, :H], a2f[:ET, :H]


# speedup vs baseline: 3.6843x; 3.6843x over previous
import functools

import jax
import jax.numpy as jnp
import numpy as np
from jax import lax
from jax.experimental import pallas as pl
from jax.experimental.pallas import tpu as pltpu
from jax.experimental.pallas import tpu_sc as plsc

N = 10000
E = 320000
ET = E + N
D_IN = 128
HID = 128
D_OUT = 64
H = 8

NC, NS = 2, 16
NW = NC * NS
K = 8
CPW = 1292
EPW = CPW * K
EP = NW * EPW
NP = 10240
RPT = NP // NS
ZR = 64


def _mesh():
    return plsc.VectorSubcoreMesh(core_axis_name="c", subcore_axis_name="s",
                                  num_cores=NC, num_subcores=NS)


def _perm(D):
    C = D // H
    p = np.empty((D,), np.int64)
    for pos in range(D):
        j, r = divmod(pos, 16)
        h = r if r < 8 else 15 - r
        c = 2 * j + (0 if r < 8 else 1)
        p[pos] = h * C + c
    return jnp.asarray(p)


def _selector(D):
    s = np.zeros((D, 16), np.float32)
    for pos in range(D):
        _, r = divmod(pos, 16)
        h = r if r < 8 else 15 - r
        s[pos, h] = 1.0
    return jnp.asarray(s)


_R = 400
_NB = N // _R


def _proj_compute(xb, wlh_ref, wrh_ref, attf_ref, sel_ref, xlh_ref, xrs_ref):
    xlh = jnp.dot(xb, wlh_ref[...], preferred_element_type=jnp.float32)
    xrh = jnp.dot(xb, wrh_ref[...], preferred_element_type=jnp.float32)
    s = xlh + xrh
    lr = jnp.where(s > 0.0, s, 0.2 * s) * attf_ref[...]
    sb = jnp.dot(lr, sel_ref[...], preferred_element_type=jnp.float32)
    R = xb.shape[0]
    nid = (pl.program_id(0) * R
           + lax.broadcasted_iota(jnp.int32, (R, 16), 0)).astype(jnp.float32)
    pad = jnp.zeros((R, 96), jnp.float32)
    xlh_ref[...] = xlh
    xrs_ref[...] = jnp.concatenate([xrh, sb, nid, pad], axis=1)


def _proj1_body(x_ref, wlh_ref, wrh_ref, attf_ref, sel_ref,
                xlh_ref, xrs_ref):
    _proj_compute(x_ref[...], wlh_ref, wrh_ref, attf_ref, sel_ref,
                  xlh_ref, xrs_ref)


def _wspecs(Cin, D):
    return [
        pl.BlockSpec((Cin, D), lambda i: (0, 0)),
        pl.BlockSpec((Cin, D), lambda i: (0, 0)),
        pl.BlockSpec((1, D), lambda i: (0, 0)),
        pl.BlockSpec((D, 16), lambda i: (0, 0)),
    ]


def _proj_outs(D):
    out_specs = [
        pl.BlockSpec((_R, D), lambda i: (i, 0)),
        pl.BlockSpec((_R, D + 128), lambda i: (i, 0)),
    ]
    out_shape = [
        jax.ShapeDtypeStruct((N, D), jnp.float32),
        jax.ShapeDtypeStruct((N, D + 128), jnp.float32),
    ]
    return out_specs, out_shape


def _proj1(x, wlh, wrh, attf, sel):
    D = wlh.shape[1]
    out_specs, out_shape = _proj_outs(D)
    return pl.pallas_call(
        _proj1_body, grid=(_NB,),
        in_specs=[pl.BlockSpec((_R, x.shape[1]), lambda i: (i, 0))]
        + _wspecs(x.shape[1], D),
        out_specs=out_specs, out_shape=out_shape,
    )(x, wlh, wrh, attf, sel)


def _proj2_body(Cin, ms0_ref, ms1_ref, b_ref, wlh_ref, wrh_ref,
                attf_ref, sel_ref, xlh_ref, xrs_ref, h1_ref):
    hsum = (ms0_ref[...] + ms1_ref[...])[:, :Cin]
    h1 = jnp.maximum(hsum * (1.0 / H) + b_ref[...], 0.0)
    h1_ref[...] = h1
    _proj_compute(h1, wlh_ref, wrh_ref, attf_ref, sel_ref, xlh_ref, xrs_ref)


def _proj2(ms0, ms1, b, wlh, wrh, attf, sel):
    Cin = b.shape[0]
    D = wlh.shape[1]
    out_specs, out_shape = _proj_outs(D)
    return pl.pallas_call(
        functools.partial(_proj2_body, Cin), grid=(_NB,),
        in_specs=[
            pl.BlockSpec((_R, 128), lambda i: (i, 0)),
            pl.BlockSpec((_R, 128), lambda i: (i, 0)),
            pl.BlockSpec((1, Cin), lambda i: (0, 0)),
        ] + _wspecs(Cin, D),
        out_specs=out_specs + [pl.BlockSpec((_R, Cin), lambda i: (i, 0))],
        out_shape=out_shape + [jax.ShapeDtypeStruct((N, Cin), jnp.float32)],
    )(ms0, ms1, b.reshape(1, -1), wlh, wrh, attf, sel)


def _msgtab_body(x_ref, w_ref, out_ref):
    out_ref[...] = jnp.dot(x_ref[...], w_ref[...],
                           preferred_element_type=jnp.float32)


def _msgtab(x, w):
    Cin = x.shape[1]
    D = w.shape[1]
    return pl.pallas_call(
        _msgtab_body, grid=(_NB,),
        in_specs=[
            pl.BlockSpec((_R, Cin), lambda i: (i, 0)),
            pl.BlockSpec((Cin, D), lambda i: (0, 0)),
        ],
        out_specs=pl.BlockSpec((_R, D), lambda i: (i, 0)),
        out_shape=jax.ShapeDtypeStruct((N, D), jnp.float32),
    )(x, w)


def _deninv_body(d0_ref, d1_ref, out_ref):
    out_ref[...] = 1.0 / (d0_ref[...] + d1_ref[...] + 1e-16)


def _deninv(d0, d1):
    B = 512
    return pl.pallas_call(
        _deninv_body, grid=(NP // B,),
        in_specs=[
            pl.BlockSpec((B, 128), lambda i: (i, 0)),
            pl.BlockSpec((B, 128), lambda i: (i, 0)),
        ],
        out_specs=pl.BlockSpec((B, 128), lambda i: (i, 0)),
        out_shape=jax.ShapeDtypeStruct((NP, 128), jnp.float32),
    )(d0, d1)


def _final_body(C, ms0_ref, ms1_ref, b_ref, out_ref):
    h2 = ((ms0_ref[...] + ms1_ref[...])[:, :C] * (1.0 / H) + b_ref[...])
    m = jnp.max(h2, axis=1, keepdims=True)
    z = h2 - m
    lse = jnp.log(jnp.sum(jnp.exp(z), axis=1, keepdims=True))
    out_ref[...] = z - lse


def _final(ms0, ms1, b):
    C = b.shape[0]
    return pl.pallas_call(
        functools.partial(_final_body, C),
        grid=(_NB,),
        in_specs=[
            pl.BlockSpec((_R, 128), lambda i: (i, 0)),
            pl.BlockSpec((_R, 128), lambda i: (i, 0)),
            pl.BlockSpec((1, C), lambda i: (0, 0)),
        ],
        out_specs=pl.BlockSpec((_R, C), lambda i: (i, 0)),
        out_shape=jax.ShapeDtypeStruct((N, C), jnp.float32),
    )(ms0, ms1, b.reshape(1, -1))


def _make_edge_pass1(D):
    NV = D // 16

    @functools.partial(
        pl.kernel,
        out_type=(
            jax.ShapeDtypeStruct((EP, 16), jnp.float32),
            jax.ShapeDtypeStruct((NP, 128), jnp.float32),
            jax.ShapeDtypeStruct((NP, 128), jnp.float32),
        ),
        mesh=_mesh(),
        scratch_types=[
            pltpu.VMEM((K,), jnp.int32),
            pltpu.VMEM((K,), jnp.int32),
            pltpu.VMEM((1, K), jnp.int32),
            pltpu.VMEM((K, D), jnp.float32),
            pltpu.VMEM((K, D + 128), jnp.float32),
            pltpu.VMEM((K, 16), jnp.float32),
            pltpu.VMEM((K, 128), jnp.float32),
            pltpu.VMEM((D,), jnp.float32),
            pltpu.VMEM((ZR, 128), jnp.float32),
            pltpu.VMEM_SHARED((NP, 128), jnp.float32),
            pltpu.SemaphoreType.DMA,
        ],
    )
    def kfn(xlh_hbm, xrs_hbm, att_hbm, src_hbm, dst_hbm,
            ex_hbm, den0_hbm, den1_hbm,
            srcv, dstv, dsti, xlr, xrr, exb, exb128, attv, zbuf, den_acc,
            sem):
        cid = lax.axis_index("c")
        sid = lax.axis_index("s")
        wid = sid * NC + cid
        r0 = sid * RPT

        def zrow(i, _):
            for j in range(8):
                zbuf[i, pl.ds(j * 16, 16)] = jnp.zeros((16,), jnp.float32)
            return 0
        lax.fori_loop(0, ZR, zrow, 0)

        def zex(i, _):
            for j in range(8):
                exb128[i, pl.ds(j * 16, 16)] = jnp.zeros((16,), jnp.float32)
            return 0
        lax.fori_loop(0, K, zex, 0)

        for j in range(RPT // ZR):
            pltpu.sync_copy(zbuf, den_acc.at[pl.ds(r0 + j * ZR, ZR)])
        pltpu.sync_copy(att_hbm, attv)
        plsc.subcore_barrier()

        def chunk(g, _):
            base = wid * EPW + g * K
            pltpu.sync_copy(src_hbm.at[pl.ds(base, K)], srcv)
            pltpu.sync_copy(dst_hbm.at[pl.ds(base, K)], dstv)
            pltpu.sync_copy(dst_hbm.at[pl.ds(base, K)], dsti.at[0])
            pltpu.async_copy(xlh_hbm.at[srcv], xlr, sem).wait()
            pltpu.async_copy(xrs_hbm.at[dstv], xrr, sem).wait()

            def edge(i, _):
                acc = jnp.zeros((16,), jnp.float32)
                for j in range(NV):
                    off = j * 16
                    s = xlr[i, pl.ds(off, 16)] + xrr[i, pl.ds(off, 16)]
                    s = 0.6 * s + 0.4 * jnp.abs(s)
                    acc = acc + s * attv[pl.ds(off, 16)]
                av = acc + jnp.flip(acc, 0)
                exv = jnp.exp(av - xrr[i, pl.ds(D, 16)])
                iot = lax.broadcasted_iota(jnp.int32, (16,), 0)
                maskf = jnp.clip(H - iot, 0, 1).astype(jnp.float32)
                validf = ((base + i) < ET).astype(jnp.float32)
                exv = exv * (maskf * validf)
                exb[i, :] = exv
                exb128[i, pl.ds(0, 16)] = exv
                return 0
            lax.fori_loop(0, K, edge, 0)
            pltpu.sync_copy(exb, ex_hbm.at[pl.ds(base, K)])
            pltpu.sync_copy(exb128, den_acc.at[dsti.at[0]], add=True)
            return 0
        lax.fori_loop(0, CPW, chunk, 0)
        plsc.subcore_barrier()

        @pl.when(cid == 0)
        def _():
            pltpu.sync_copy(den_acc.at[pl.ds(r0, RPT)],
                            den0_hbm.at[pl.ds(r0, RPT)])

        @pl.when(cid == 1)
        def _():
            pltpu.sync_copy(den_acc.at[pl.ds(r0, RPT)],
                            den1_hbm.at[pl.ds(r0, RPT)])

    return kfn


def _make_edge_pass2(D):
    C = D // H
    CB = C // 16

    @functools.partial(
        pl.kernel,
        out_type=(
            jax.ShapeDtypeStruct((EP, 16), jnp.float32),
            jax.ShapeDtypeStruct((NP, 128), jnp.float32),
            jax.ShapeDtypeStruct((NP, 128), jnp.float32),
        ),
        mesh=_mesh(),
        scratch_types=[
            pltpu.VMEM((K,), jnp.int32),
            pltpu.VMEM((K,), jnp.int32),
            pltpu.VMEM((1, K), jnp.int32),
            pltpu.VMEM((K, D), jnp.float32),
            pltpu.VMEM((K, 16), jnp.float32),
            pltpu.VMEM((K, 128), jnp.float32),
            pltpu.VMEM((K, 16), jnp.float32),
            pltpu.VMEM((K, 128), jnp.float32),
            pltpu.VMEM((ZR, 128), jnp.float32),
            pltpu.VMEM_SHARED((NP, 128), jnp.float32),
            pltpu.SemaphoreType.DMA,
        ],
    )
    def kfn(xm_hbm, ex_hbm, dinv_hbm, src_hbm, dst_hbm,
            a_hbm, ms0_hbm, ms1_hbm,
            srcv, dstv, dsti, xlr, exb, dvr, ab, mb, zbuf, m_acc, sem):
        cid = lax.axis_index("c")
        sid = lax.axis_index("s")
        wid = sid * NC + cid
        r0 = sid * RPT

        def zrow(i, _):
            for j in range(8):
                zbuf[i, pl.ds(j * 16, 16)] = jnp.zeros((16,), jnp.float32)
            return 0
        lax.fori_loop(0, ZR, zrow, 0)

        def zmb(i, _):
            for j in range(8):
                mb[i, pl.ds(j * 16, 16)] = jnp.zeros((16,), jnp.float32)
            return 0
        lax.fori_loop(0, K, zmb, 0)

        for j in range(RPT // ZR):
            pltpu.sync_copy(zbuf, m_acc.at[pl.ds(r0 + j * ZR, ZR)])
        plsc.subcore_barrier()

        def chunk(g, _):
            base = wid * EPW + g * K
            pltpu.sync_copy(src_hbm.at[pl.ds(base, K)], srcv)
            pltpu.sync_copy(dst_hbm.at[pl.ds(base, K)], dstv)
            pltpu.sync_copy(dst_hbm.at[pl.ds(base, K)], dsti.at[0])
            pltpu.async_copy(xm_hbm.at[srcv], xlr, sem).wait()
            pltpu.sync_copy(ex_hbm.at[pl.ds(base, K)], exb)
            pltpu.async_copy(dinv_hbm.at[dstv], dvr, sem).wait()

            def edge(i, _):
                a = exb[i, :] * dvr[i, pl.ds(0, 16)]
                ab[i, :] = a
                for j in range(CB):
                    acc = jnp.zeros((16,), jnp.float32)
                    for h in range(H):
                        off = h * C + j * 16
                        acc = acc + a[h] * xlr[i, pl.ds(off, 16)]
                    mb[i, pl.ds(j * 16, 16)] = acc
                return 0
            lax.fori_loop(0, K, edge, 0)
            pltpu.sync_copy(ab, a_hbm.at[pl.ds(base, K)])
            pltpu.sync_copy(mb, m_acc.at[dsti.at[0]], add=True)
            return 0
        lax.fori_loop(0, CPW, chunk, 0)
        plsc.subcore_barrier()

        @pl.when(cid == 0)
        def _():
            pltpu.sync_copy(m_acc.at[pl.ds(r0, RPT)],
                            ms0_hbm.at[pl.ds(r0, RPT)])

        @pl.when(cid == 1)
        def _():
            pltpu.sync_copy(m_acc.at[pl.ds(r0, RPT)],
                            ms1_hbm.at[pl.ds(r0, RPT)])

    return kfn


_edge1 = functools.lru_cache(maxsize=None)(_make_edge_pass1)
_edge2 = functools.lru_cache(maxsize=None)(_make_edge_pass2)


def kernel(x, edge_index, Wl1, Wr1, att1, b1, Wl2, Wr2, att2, b2):
    loops = jnp.arange(N, dtype=edge_index.dtype)
    ei = jnp.concatenate([edge_index, jnp.stack([loops, loops])], axis=1)
    src = jnp.pad(ei[0], (0, EP - ET)).astype(jnp.int32)
    dst = jnp.pad(ei[1], (0, EP - ET)).astype(jnp.int32)

    D1 = H * HID
    p1 = _perm(D1)
    att1p = att1.reshape(-1)[p1]
    xlh1, xrs1 = _proj1(x, Wl1[:, p1], Wr1[:, p1],
                        att1p.reshape(1, -1), _selector(D1))
    xm1 = _msgtab(x, Wl1)
    ex1, d10, d11 = _edge1(D1)(xlh1, xrs1, att1p, src, dst)
    dinv1 = _deninv(d10, d11)
    a1f, ms10, ms11 = _edge2(D1)(xm1, ex1, dinv1, src, dst)

    D2 = H * D_OUT
    p2 = _perm(D2)
    att2p = att2.reshape(-1)[p2]
    xlh2, xrs2, h1 = _proj2(ms10, ms11, b1, Wl2[:, p2], Wr2[:, p2],
                            att2p.reshape(1, -1), _selector(D2))
    xm2 = _msgtab(h1, Wl2)
    ex2, d20, d21 = _edge1(D2)(xlh2, xrs2, att2p, src, dst)
    dinv2 = _deninv(d20, d21)
    a2f, ms20, ms21 = _edge2(D2)(xm2, ex2, dinv2, src, dst)

    out = _final(ms20, ms21, b2)
    return out, ei, a1f[:ET, :H], a2f[:ET, :H]


# speedup vs baseline: 5.7887x; 1.5712x over previous
import functools

import jax
import jax.numpy as jnp
import numpy as np
from jax import lax
from jax.experimental import pallas as pl
from jax.experimental.pallas import tpu as pltpu
from jax.experimental.pallas import tpu_sc as plsc

N = 10000
E = 320000
ET = E + N
D_IN = 128
HID = 128
D_OUT = 64
H = 8

NC, NS = 2, 16
NW = NC * NS
K = 16
CPW = 646
EPW = CPW * K
EP = NW * EPW
NP = 10240
RPT = NP // NS
ZR = 32


def _mesh():
    return plsc.VectorSubcoreMesh(core_axis_name="c", subcore_axis_name="s",
                                  num_cores=NC, num_subcores=NS)


def _perm(D):
    C = D // H
    p = np.empty((D,), np.int64)
    for pos in range(D):
        j, r = divmod(pos, 16)
        h = r if r < 8 else 15 - r
        c = 2 * j + (0 if r < 8 else 1)
        p[pos] = h * C + c
    return jnp.asarray(p)


def _selector(D):
    s = np.zeros((D, 16), np.float32)
    for pos in range(D):
        _, r = divmod(pos, 16)
        h = r if r < 8 else 15 - r
        s[pos, h] = 1.0
    return jnp.asarray(s)


_R = 400
_NB = N // _R


def _proj_compute(xb, wlh_ref, wrh_ref, attf_ref, sel_ref, xlh_ref, xrs_ref):
    xlh = jnp.dot(xb, wlh_ref[...], preferred_element_type=jnp.float32)
    xrh = jnp.dot(xb, wrh_ref[...], preferred_element_type=jnp.float32)
    s = xlh + xrh
    lr = jnp.where(s > 0.0, s, 0.2 * s) * attf_ref[...]
    sb = jnp.dot(lr, sel_ref[...], preferred_element_type=jnp.float32)
    R = xb.shape[0]
    nid = (pl.program_id(0) * R
           + lax.broadcasted_iota(jnp.int32, (R, 16), 0)).astype(jnp.float32)
    pad = jnp.zeros((R, 96), jnp.float32)
    xlh_ref[...] = xlh
    xrs_ref[...] = jnp.concatenate([xrh, sb, nid, pad], axis=1)


def _proj1_body(x_ref, wlh_ref, wrh_ref, attf_ref, sel_ref,
                xlh_ref, xrs_ref):
    _proj_compute(x_ref[...], wlh_ref, wrh_ref, attf_ref, sel_ref,
                  xlh_ref, xrs_ref)


def _wspecs(Cin, D):
    return [
        pl.BlockSpec((Cin, D), lambda i: (0, 0)),
        pl.BlockSpec((Cin, D), lambda i: (0, 0)),
        pl.BlockSpec((1, D), lambda i: (0, 0)),
        pl.BlockSpec((D, 16), lambda i: (0, 0)),
    ]


def _proj_outs(D):
    out_specs = [
        pl.BlockSpec((_R, D), lambda i: (i, 0)),
        pl.BlockSpec((_R, D + 128), lambda i: (i, 0)),
    ]
    out_shape = [
        jax.ShapeDtypeStruct((N, D), jnp.float32),
        jax.ShapeDtypeStruct((N, D + 128), jnp.float32),
    ]
    return out_specs, out_shape


def _proj1(x, wlh, wrh, attf, sel):
    D = wlh.shape[1]
    out_specs, out_shape = _proj_outs(D)
    return pl.pallas_call(
        _proj1_body, grid=(_NB,),
        in_specs=[pl.BlockSpec((_R, x.shape[1]), lambda i: (i, 0))]
        + _wspecs(x.shape[1], D),
        out_specs=out_specs, out_shape=out_shape,
    )(x, wlh, wrh, attf, sel)


def _proj2_body(Cin, ms0_ref, ms1_ref, b_ref, wlh_ref, wrh_ref,
                attf_ref, sel_ref, xlh_ref, xrs_ref, h1_ref):
    hsum = (ms0_ref[...] + ms1_ref[...])[:, :Cin]
    h1 = jnp.maximum(hsum * (1.0 / H) + b_ref[...], 0.0)
    h1_ref[...] = h1
    _proj_compute(h1, wlh_ref, wrh_ref, attf_ref, sel_ref, xlh_ref, xrs_ref)


def _proj2(ms0, ms1, b, wlh, wrh, attf, sel):
    Cin = b.shape[0]
    D = wlh.shape[1]
    out_specs, out_shape = _proj_outs(D)
    return pl.pallas_call(
        functools.partial(_proj2_body, Cin), grid=(_NB,),
        in_specs=[
            pl.BlockSpec((_R, 128), lambda i: (i, 0)),
            pl.BlockSpec((_R, 128), lambda i: (i, 0)),
            pl.BlockSpec((1, Cin), lambda i: (0, 0)),
        ] + _wspecs(Cin, D),
        out_specs=out_specs + [pl.BlockSpec((_R, Cin), lambda i: (i, 0))],
        out_shape=out_shape + [jax.ShapeDtypeStruct((N, Cin), jnp.float32)],
    )(ms0, ms1, b.reshape(1, -1), wlh, wrh, attf, sel)


def _msgtab_body(x_ref, w_ref, out_ref):
    out_ref[...] = jnp.dot(x_ref[...], w_ref[...],
                           preferred_element_type=jnp.float32)


def _msgtab(x, w):
    Cin = x.shape[1]
    D = w.shape[1]
    return pl.pallas_call(
        _msgtab_body, grid=(_NB,),
        in_specs=[
            pl.BlockSpec((_R, Cin), lambda i: (i, 0)),
            pl.BlockSpec((Cin, D), lambda i: (0, 0)),
        ],
        out_specs=pl.BlockSpec((_R, D), lambda i: (i, 0)),
        out_shape=jax.ShapeDtypeStruct((N, D), jnp.float32),
    )(x, w)


def _deninv_body(d0_ref, d1_ref, out_ref):
    out_ref[...] = 1.0 / (d0_ref[...] + d1_ref[...] + 1e-16)


def _deninv(d0, d1):
    B = 512
    return pl.pallas_call(
        _deninv_body, grid=(NP // B,),
        in_specs=[
            pl.BlockSpec((B, 128), lambda i: (i, 0)),
            pl.BlockSpec((B, 128), lambda i: (i, 0)),
        ],
        out_specs=pl.BlockSpec((B, 128), lambda i: (i, 0)),
        out_shape=jax.ShapeDtypeStruct((NP, 128), jnp.float32),
    )(d0, d1)


def _final_body(C, ms0_ref, ms1_ref, b_ref, out_ref):
    h2 = ((ms0_ref[...] + ms1_ref[...])[:, :C] * (1.0 / H) + b_ref[...])
    m = jnp.max(h2, axis=1, keepdims=True)
    z = h2 - m
    lse = jnp.log(jnp.sum(jnp.exp(z), axis=1, keepdims=True))
    out_ref[...] = z - lse


def _final(ms0, ms1, b):
    C = b.shape[0]
    return pl.pallas_call(
        functools.partial(_final_body, C),
        grid=(_NB,),
        in_specs=[
            pl.BlockSpec((_R, 128), lambda i: (i, 0)),
            pl.BlockSpec((_R, 128), lambda i: (i, 0)),
            pl.BlockSpec((1, C), lambda i: (0, 0)),
        ],
        out_specs=pl.BlockSpec((_R, C), lambda i: (i, 0)),
        out_shape=jax.ShapeDtypeStruct((N, C), jnp.float32),
    )(ms0, ms1, b.reshape(1, -1))


def _make_edge_pass1(D):
    NV = D // 16

    @functools.partial(
        pl.kernel,
        out_type=(
            jax.ShapeDtypeStruct((EP, 16), jnp.float32),
            jax.ShapeDtypeStruct((NP, 128), jnp.float32),
            jax.ShapeDtypeStruct((NP, 128), jnp.float32),
        ),
        mesh=_mesh(),
        scratch_types=[
            pltpu.VMEM((K,), jnp.int32),
            pltpu.VMEM((K,), jnp.int32),
            pltpu.VMEM((1, K), jnp.int32),
            pltpu.VMEM((K, D), jnp.float32),
            pltpu.VMEM((K, D + 128), jnp.float32),
            pltpu.VMEM((K, 16), jnp.float32),
            pltpu.VMEM((K, 128), jnp.float32),
            pltpu.VMEM((D,), jnp.float32),
            pltpu.VMEM((ZR, 128), jnp.float32),
            pltpu.VMEM_SHARED((NP, 128), jnp.float32),
            pltpu.SemaphoreType.DMA,
        ],
    )
    def kfn(xlh_hbm, xrs_hbm, att_hbm, src_hbm, dst_hbm,
            ex_hbm, den0_hbm, den1_hbm,
            srcv, dstv, dsti, xlr, xrr, exb, exb128, attv, zbuf, den_acc,
            sem):
        cid = lax.axis_index("c")
        sid = lax.axis_index("s")
        wid = sid * NC + cid
        r0 = sid * RPT

        def zrow(i, _):
            for j in range(8):
                zbuf[i, pl.ds(j * 16, 16)] = jnp.zeros((16,), jnp.float32)
            return 0
        lax.fori_loop(0, ZR, zrow, 0)

        def zex(i, _):
            for j in range(8):
                exb128[i, pl.ds(j * 16, 16)] = jnp.zeros((16,), jnp.float32)
            return 0
        lax.fori_loop(0, K, zex, 0)

        for j in range(RPT // ZR):
            pltpu.sync_copy(zbuf, den_acc.at[pl.ds(r0 + j * ZR, ZR)])
        pltpu.sync_copy(att_hbm, attv)
        plsc.subcore_barrier()

        def chunk(g, _):
            base = wid * EPW + g * K
            pltpu.sync_copy(src_hbm.at[pl.ds(base, K)], srcv)
            pltpu.sync_copy(dst_hbm.at[pl.ds(base, K)], dstv)
            pltpu.sync_copy(dst_hbm.at[pl.ds(base, K)], dsti.at[0])
            pltpu.async_copy(xlh_hbm.at[srcv], xlr, sem).wait()
            pltpu.async_copy(xrs_hbm.at[dstv], xrr, sem).wait()

            def edge(i, _):
                acc = jnp.zeros((16,), jnp.float32)
                for j in range(NV):
                    off = j * 16
                    s = xlr[i, pl.ds(off, 16)] + xrr[i, pl.ds(off, 16)]
                    s = 0.6 * s + 0.4 * jnp.abs(s)
                    acc = acc + s * attv[pl.ds(off, 16)]
                av = acc + jnp.flip(acc, 0)
                exv = jnp.exp(av - xrr[i, pl.ds(D, 16)])
                iot = lax.broadcasted_iota(jnp.int32, (16,), 0)
                maskf = jnp.clip(H - iot, 0, 1).astype(jnp.float32)
                validf = ((base + i) < ET).astype(jnp.float32)
                exv = exv * (maskf * validf)
                exb[i, :] = exv
                exb128[i, pl.ds(0, 16)] = exv
                return 0
            lax.fori_loop(0, K, edge, 0)
            pltpu.sync_copy(exb, ex_hbm.at[pl.ds(base, K)])
            pltpu.sync_copy(exb128, den_acc.at[dsti.at[0]], add=True)
            return 0
        lax.fori_loop(0, CPW, chunk, 0)
        plsc.subcore_barrier()

        @pl.when(cid == 0)
        def _():
            pltpu.sync_copy(den_acc.at[pl.ds(r0, RPT)],
                            den0_hbm.at[pl.ds(r0, RPT)])

        @pl.when(cid == 1)
        def _():
            pltpu.sync_copy(den_acc.at[pl.ds(r0, RPT)],
                            den1_hbm.at[pl.ds(r0, RPT)])

    return kfn


def _make_edge_pass2(D):
    C = D // H
    CB = C // 16

    @functools.partial(
        pl.kernel,
        out_type=(
            jax.ShapeDtypeStruct((EP, 16), jnp.float32),
            jax.ShapeDtypeStruct((NP, 128), jnp.float32),
            jax.ShapeDtypeStruct((NP, 128), jnp.float32),
        ),
        mesh=_mesh(),
        scratch_types=[
            pltpu.VMEM((K,), jnp.int32),
            pltpu.VMEM((K,), jnp.int32),
            pltpu.VMEM((1, K), jnp.int32),
            pltpu.VMEM((K, D), jnp.float32),
            pltpu.VMEM((K, 16), jnp.float32),
            pltpu.VMEM((K, 128), jnp.float32),
            pltpu.VMEM((K, 16), jnp.float32),
            pltpu.VMEM((K, 128), jnp.float32),
            pltpu.VMEM((ZR, 128), jnp.float32),
            pltpu.VMEM_SHARED((NP, 128), jnp.float32),
            pltpu.SemaphoreType.DMA,
        ],
    )
    def kfn(xm_hbm, ex_hbm, dinv_hbm, src_hbm, dst_hbm,
            a_hbm, ms0_hbm, ms1_hbm,
            srcv, dstv, dsti, xlr, exb, dvr, ab, mb, zbuf, m_acc, sem):
        cid = lax.axis_index("c")
        sid = lax.axis_index("s")
        wid = sid * NC + cid
        r0 = sid * RPT

        def zrow(i, _):
            for j in range(8):
                zbuf[i, pl.ds(j * 16, 16)] = jnp.zeros((16,), jnp.float32)
            return 0
        lax.fori_loop(0, ZR, zrow, 0)

        def zmb(i, _):
            for j in range(8):
                mb[i, pl.ds(j * 16, 16)] = jnp.zeros((16,), jnp.float32)
            return 0
        lax.fori_loop(0, K, zmb, 0)

        for j in range(RPT // ZR):
            pltpu.sync_copy(zbuf, m_acc.at[pl.ds(r0 + j * ZR, ZR)])
        plsc.subcore_barrier()

        def chunk(g, _):
            base = wid * EPW + g * K
            pltpu.sync_copy(src_hbm.at[pl.ds(base, K)], srcv)
            pltpu.sync_copy(dst_hbm.at[pl.ds(base, K)], dstv)
            pltpu.sync_copy(dst_hbm.at[pl.ds(base, K)], dsti.at[0])
            pltpu.async_copy(xm_hbm.at[srcv], xlr, sem).wait()
            pltpu.sync_copy(ex_hbm.at[pl.ds(base, K)], exb)
            pltpu.async_copy(dinv_hbm.at[dstv], dvr, sem).wait()

            def edge(i, _):
                a = exb[i, :] * dvr[i, pl.ds(0, 16)]
                ab[i, :] = a
                for j in range(CB):
                    acc = jnp.zeros((16,), jnp.float32)
                    for h in range(H):
                        off = h * C + j * 16
                        acc = acc + a[h] * xlr[i, pl.ds(off, 16)]
                    mb[i, pl.ds(j * 16, 16)] = acc
                return 0
            lax.fori_loop(0, K, edge, 0)
            pltpu.sync_copy(ab, a_hbm.at[pl.ds(base, K)])
            pltpu.sync_copy(mb, m_acc.at[dsti.at[0]], add=True)
            return 0
        lax.fori_loop(0, CPW, chunk, 0)
        plsc.subcore_barrier()

        @pl.when(cid == 0)
        def _():
            pltpu.sync_copy(m_acc.at[pl.ds(r0, RPT)],
                            ms0_hbm.at[pl.ds(r0, RPT)])

        @pl.when(cid == 1)
        def _():
            pltpu.sync_copy(m_acc.at[pl.ds(r0, RPT)],
                            ms1_hbm.at[pl.ds(r0, RPT)])

    return kfn


_edge1 = functools.lru_cache(maxsize=None)(_make_edge_pass1)
_edge2 = functools.lru_cache(maxsize=None)(_make_edge_pass2)


def kernel(x, edge_index, Wl1, Wr1, att1, b1, Wl2, Wr2, att2, b2):
    loops = jnp.arange(N, dtype=edge_index.dtype)
    ei = jnp.concatenate([edge_index, jnp.stack([loops, loops])], axis=1)
    src = jnp.pad(ei[0], (0, EP - ET)).astype(jnp.int32)
    dst = jnp.pad(ei[1], (0, EP - ET)).astype(jnp.int32)

    D1 = H * HID
    p1 = _perm(D1)
    att1p = att1.reshape(-1)[p1]
    xlh1, xrs1 = _proj1(x, Wl1[:, p1], Wr1[:, p1],
                        att1p.reshape(1, -1), _selector(D1))
    xm1 = _msgtab(x, Wl1)
    ex1, d10, d11 = _edge1(D1)(xlh1, xrs1, att1p, src, dst)
    dinv1 = _deninv(d10, d11)
    a1f, ms10, ms11 = _edge2(D1)(xm1, ex1, dinv1, src, dst)

    D2 = H * D_OUT
    p2 = _perm(D2)
    att2p = att2.reshape(-1)[p2]
    xlh2, xrs2, h1 = _proj2(ms10, ms11, b1, Wl2[:, p2], Wr2[:, p2],
                            att2p.reshape(1, -1), _selector(D2))
    xm2 = _msgtab(h1, Wl2)
    ex2, d20, d21 = _edge1(D2)(xlh2, xrs2, att2p, src, dst)
    dinv2 = _deninv(d20, d21)
    a2f, ms20, ms21 = _edge2(D2)(xm2, ex2, dinv2, src, dst)

    out = _final(ms20, ms21, b2)
    return out, ei, a1f[:ET, :H], a2f[:ET, :H]
